# Initial kernel scaffold; baseline (speedup 1.0000x reference)
import jax
import jax.numpy as jnp
from jax.experimental import pallas as pl


def kernel(indices, embeddings):
    raise NotImplementedError("write your pallas kernel here")


# speedup vs baseline: 1.0110x; 1.0110x over previous
import jax
import jax.numpy as jnp
from jax.experimental import pallas as pl
from jax.experimental.pallas import tpu as pltpu

_NUM_ACTIONS = 27
_MAX_LENGTH = 77
_T5_DIM = 4096
_D = _MAX_LENGTH * _T5_DIM
_ROWS_PER_BLOCK = 8


def _gather_body(idx_ref, emb_ref, out_ref):
    i = pl.program_id(0)
    for j in range(_ROWS_PER_BLOCK):
        a = idx_ref[i * _ROWS_PER_BLOCK + j]
        out_ref[pl.ds(j, 1), :] = emb_ref[pl.ds(a, 1), :]


def kernel(indices, embeddings):
    batch = indices.shape[0]
    emb2 = embeddings.reshape(_NUM_ACTIONS, _D)
    out = pl.pallas_call(
        _gather_body,
        grid_spec=pltpu.PrefetchScalarGridSpec(
            num_scalar_prefetch=1,
            grid=(batch // _ROWS_PER_BLOCK,),
            in_specs=[
                pl.BlockSpec((_NUM_ACTIONS, _D), lambda i, idx_ref: (0, 0)),
            ],
            out_specs=pl.BlockSpec((_ROWS_PER_BLOCK, _D), lambda i, idx_ref: (i, 0)),
        ),
        out_shape=jax.ShapeDtypeStruct((batch, _D), jnp.float32),
    )(indices.astype(jnp.int32), emb2)
    return out.reshape(batch, _MAX_LENGTH, _T5_DIM)


# speedup vs baseline: 2.0951x; 2.0723x over previous
import jax
import jax.numpy as jnp
from jax.experimental import pallas as pl
from jax.experimental.pallas import tpu as pltpu

_NUM_ACTIONS = 27
_MAX_LENGTH = 77
_T5_DIM = 4096
_ROWS_PER_BLOCK = 4


def _gather_body(idx_ref, emb_ref, out_ref):
    i = pl.program_id(0)
    for j in range(_ROWS_PER_BLOCK):
        a = idx_ref[i * _ROWS_PER_BLOCK + j]
        out_ref[pl.ds(j, 1)] = emb_ref[pl.ds(a, 1)]


def kernel(indices, embeddings):
    batch = indices.shape[0]
    out = pl.pallas_call(
        _gather_body,
        grid_spec=pltpu.PrefetchScalarGridSpec(
            num_scalar_prefetch=1,
            grid=(batch // _ROWS_PER_BLOCK,),
            in_specs=[
                pl.BlockSpec(
                    (_NUM_ACTIONS, _MAX_LENGTH, _T5_DIM), lambda i, idx_ref: (0, 0, 0)
                ),
            ],
            out_specs=pl.BlockSpec(
                (_ROWS_PER_BLOCK, _MAX_LENGTH, _T5_DIM), lambda i, idx_ref: (i, 0, 0)
            ),
        ),
        out_shape=jax.ShapeDtypeStruct((batch, _MAX_LENGTH, _T5_DIM), jnp.float32),
    )(indices.astype(jnp.int32), embeddings)
    return out


# speedup vs baseline: 2.0958x; 1.0003x over previous
import jax
import jax.numpy as jnp
from jax.experimental import pallas as pl
from jax.experimental.pallas import tpu as pltpu

_NUM_ACTIONS = 27
_MAX_LENGTH = 77
_T5_DIM = 4096
_NSEM = 8


def _dma_body(idx_ref, emb_hbm, out_hbm, emb_vmem, sem_t, sems):
    batch = out_hbm.shape[0]
    pltpu.make_async_copy(emb_hbm, emb_vmem, sem_t).start()
    pltpu.make_async_copy(emb_hbm, emb_vmem, sem_t).wait()

    def _copy(i, k):
        return pltpu.make_async_copy(
            emb_vmem.at[idx_ref[i]], out_hbm.at[i], sems.at[k]
        )

    for k in range(_NSEM):
        _copy(k, k).start()

    def _step(g, carry):
        for k in range(_NSEM):
            i = g * _NSEM + k
            _copy(i - _NSEM, k).wait()
            _copy(i, k).start()
        return carry

    jax.lax.fori_loop(1, batch // _NSEM, _step, 0)

    for k in range(_NSEM):
        _copy(batch - _NSEM + k, k).wait()


def kernel(indices, embeddings):
    batch = indices.shape[0]
    out = pl.pallas_call(
        _dma_body,
        grid_spec=pltpu.PrefetchScalarGridSpec(
            num_scalar_prefetch=1,
            grid=(1,),
            in_specs=[pl.BlockSpec(memory_space=pl.ANY)],
            out_specs=pl.BlockSpec(memory_space=pl.ANY),
            scratch_shapes=[
                pltpu.VMEM((_NUM_ACTIONS, _MAX_LENGTH, _T5_DIM), jnp.float32),
                pltpu.SemaphoreType.DMA,
                pltpu.SemaphoreType.DMA((_NSEM,)),
            ],
        ),
        out_shape=jax.ShapeDtypeStruct((batch, _MAX_LENGTH, _T5_DIM), jnp.float32),
    )(indices.astype(jnp.int32), embeddings)
    return out
